# Initial kernel scaffold; baseline (speedup 1.0000x reference)
import jax
import jax.numpy as jnp
from jax.experimental import pallas as pl


def kernel(input, table):
    raise NotImplementedError("write your pallas kernel here")


# speedup vs baseline: 1.8651x; 1.8651x over previous
import functools

import jax
import jax.numpy as jnp
from jax import lax
from jax.experimental import pallas as pl
from jax.experimental.pallas import tpu as pltpu
from jax.experimental.pallas import tpu_sc as plsc


def _build_gather(N, V, D, NC, NS):
    NW = NC * NS
    b_per_w = N // NW
    assert N % NW == 0
    CH = 1600
    assert b_per_w % CH == 0 and CH % 8 == 0
    nch = b_per_w // CH

    mesh = plsc.VectorSubcoreMesh(core_axis_name="c", subcore_axis_name="s")

    @functools.partial(
        pl.kernel,
        mesh=mesh,
        compiler_params=pltpu.CompilerParams(use_tc_tiling_on_sc=False),
        out_type=jax.ShapeDtypeStruct((N, D), jnp.float32),
        scratch_types=[
            pltpu.VMEM((CH,), jnp.int32),
            pltpu.VMEM((CH, D), jnp.float32),
            pltpu.SemaphoreType.DMA,
        ],
    )
    def gath(idx_hbm, table_hbm, out_hbm, idx_v, rows_v, sem):
        wid = lax.axis_index("s") * NC + lax.axis_index("c")
        base = wid * b_per_w

        @pl.loop(0, nch)
        def _chunk(g):
            off = pl.multiple_of(base + g * CH, 8)
            pltpu.sync_copy(idx_hbm.at[pl.ds(off, CH)], idx_v)
            pltpu.async_copy(table_hbm.at[idx_v], rows_v, sem).wait()
            pltpu.sync_copy(rows_v, out_hbm.at[pl.ds(off, CH)])

    return gath


def kernel(input, table):
    B, L = input.shape
    V, D = table.shape
    idx = input.reshape(B * L).astype(jnp.int32)
    try:
        info = plsc.get_sparse_core_info()
        NC, NS = info.num_cores, info.num_subcores
    except Exception:
        NC, NS = 2, 16
    out = _build_gather(B * L, V, D, NC, NS)(idx, table)
    return out.reshape(B, L, D)


# speedup vs baseline: 1.8737x; 1.0046x over previous
import functools

import jax
import jax.numpy as jnp
from jax import lax
from jax.experimental import pallas as pl
from jax.experimental.pallas import tpu as pltpu
from jax.experimental.pallas import tpu_sc as plsc


def _build_gather(N, V, D, NC, NS):
    NW = NC * NS
    b_per_w = N // NW
    assert N % NW == 0
    NBUF = 2
    CH = 800
    assert b_per_w % CH == 0 and CH % 8 == 0
    nch = b_per_w // CH
    assert nch % NBUF == 0 and nch > NBUF

    mesh = plsc.VectorSubcoreMesh(core_axis_name="c", subcore_axis_name="s")

    @functools.partial(
        pl.kernel,
        mesh=mesh,
        compiler_params=pltpu.CompilerParams(use_tc_tiling_on_sc=False),
        out_type=jax.ShapeDtypeStruct((N, D), jnp.float32),
        scratch_types=[
            pltpu.VMEM((b_per_w,), jnp.int32),
            pltpu.VMEM((NBUF, CH, D), jnp.float32),
        ]
        + [pltpu.SemaphoreType.DMA] * (2 * NBUF),
    )
    def gath(idx_hbm, table_hbm, out_hbm, idx_v, rows_v, *sems):
        semg = sems[:NBUF]
        semo = sems[NBUF:]
        wid = lax.axis_index("s") * NC + lax.axis_index("c")
        base = wid * b_per_w

        pltpu.sync_copy(idx_hbm.at[pl.ds(pl.multiple_of(base, 8), b_per_w)], idx_v)

        def issue(g, b):
            loff = pl.multiple_of(g * CH, 8)
            pltpu.async_copy(
                table_hbm.at[idx_v.at[pl.ds(loff, CH)]], rows_v.at[b], semg[b]
            )

        def out_slice(g):
            return out_hbm.at[pl.ds(pl.multiple_of(base + g * CH, 8), CH)]

        for b in range(NBUF - 1):
            issue(b, b)

        @pl.loop(0, nch, step=NBUF)
        def _outer(g0):
            for j in range(NBUF):
                gg = g0 + j
                b = j
                bi = (j - 1) % NBUF
                gi = gg + NBUF - 1

                @pl.when(gi < nch)
                def _issue_next():
                    def wait_prev_out():
                        pltpu.make_async_copy(
                            rows_v.at[bi], out_slice(gg - 1), semo[bi]
                        ).wait()

                    if j == 0:
                        @pl.when(g0 >= 1)
                        def _():
                            wait_prev_out()
                    else:
                        wait_prev_out()
                    issue(gi, bi)

                pltpu.make_async_copy(
                    table_hbm.at[idx_v.at[pl.ds(pl.multiple_of(gg * CH, 8), CH)]],
                    rows_v.at[b],
                    semg[b],
                ).wait()
                pltpu.async_copy(rows_v.at[b], out_slice(gg), semo[b])

        for j in range(NBUF):
            gg = nch - NBUF + j
            pltpu.make_async_copy(rows_v.at[j], out_slice(gg), semo[j]).wait()

    return gath


def kernel(input, table):
    B, L = input.shape
    V, D = table.shape
    idx = input.reshape(B * L).astype(jnp.int32)
    try:
        info = plsc.get_sparse_core_info()
        NC, NS = info.num_cores, info.num_subcores
    except Exception:
        NC, NS = 2, 16
    out = _build_gather(B * L, V, D, NC, NS)(idx, table)
    return out.reshape(B, L, D)


# speedup vs baseline: 1.8774x; 1.0019x over previous
import functools

import jax
import jax.numpy as jnp
from jax import lax
from jax.experimental import pallas as pl
from jax.experimental.pallas import tpu as pltpu
from jax.experimental.pallas import tpu_sc as plsc


def _build_gather(N, V, D, NC, NS):
    NW = NC * NS
    b_per_w = N // NW
    assert N % NW == 0
    NBUF = 4
    CH = 400
    assert b_per_w % CH == 0 and CH % 8 == 0
    nch = b_per_w // CH
    assert nch % NBUF == 0 and nch > NBUF

    mesh = plsc.VectorSubcoreMesh(core_axis_name="c", subcore_axis_name="s")

    @functools.partial(
        pl.kernel,
        mesh=mesh,
        compiler_params=pltpu.CompilerParams(use_tc_tiling_on_sc=False),
        out_type=jax.ShapeDtypeStruct((N, D), jnp.float32),
        scratch_types=[
            pltpu.VMEM((b_per_w,), jnp.int32),
            pltpu.VMEM((NBUF, CH, D), jnp.float32),
        ]
        + [pltpu.SemaphoreType.DMA] * (2 * NBUF),
    )
    def gath(idx_hbm, table_hbm, out_hbm, idx_v, rows_v, *sems):
        semg = sems[:NBUF]
        semo = sems[NBUF:]
        wid = lax.axis_index("s") * NC + lax.axis_index("c")
        base = wid * b_per_w

        pltpu.sync_copy(idx_hbm.at[pl.ds(pl.multiple_of(base, 8), b_per_w)], idx_v)

        def issue(g, b):
            loff = pl.multiple_of(g * CH, 8)
            pltpu.async_copy(
                table_hbm.at[idx_v.at[pl.ds(loff, CH)]], rows_v.at[b], semg[b]
            )

        def out_slice(g):
            return out_hbm.at[pl.ds(pl.multiple_of(base + g * CH, 8), CH)]

        for b in range(NBUF - 1):
            issue(b, b)

        @pl.loop(0, nch, step=NBUF)
        def _outer(g0):
            for j in range(NBUF):
                gg = g0 + j
                b = j
                bi = (j - 1) % NBUF
                gi = gg + NBUF - 1

                @pl.when(gi < nch)
                def _issue_next():
                    def wait_prev_out():
                        pltpu.make_async_copy(
                            rows_v.at[bi], out_slice(gg - 1), semo[bi]
                        ).wait()

                    if j == 0:
                        @pl.when(g0 >= 1)
                        def _():
                            wait_prev_out()
                    else:
                        wait_prev_out()
                    issue(gi, bi)

                pltpu.make_async_copy(
                    table_hbm.at[idx_v.at[pl.ds(pl.multiple_of(gg * CH, 8), CH)]],
                    rows_v.at[b],
                    semg[b],
                ).wait()
                pltpu.async_copy(rows_v.at[b], out_slice(gg), semo[b])

        for j in range(NBUF):
            gg = nch - NBUF + j
            pltpu.make_async_copy(rows_v.at[j], out_slice(gg), semo[j]).wait()

    return gath


def kernel(input, table):
    B, L = input.shape
    V, D = table.shape
    idx = input.reshape(B * L).astype(jnp.int32)
    try:
        info = plsc.get_sparse_core_info()
        NC, NS = info.num_cores, info.num_subcores
    except Exception:
        NC, NS = 2, 16
    out = _build_gather(B * L, V, D, NC, NS)(idx, table)
    return out.reshape(B, L, D)
